# Initial kernel scaffold; baseline (speedup 1.0000x reference)
import jax
import jax.numpy as jnp
from jax.experimental import pallas as pl


def kernel(x, Wc, bc, Wf, bf):
    raise NotImplementedError("write your pallas kernel here")


# speedup vs baseline: 3.8359x; 3.8359x over previous
import functools
import numpy as np
import jax
import jax.numpy as jnp
from jax.experimental import pallas as pl
from jax.experimental.pallas import tpu as pltpu

IN_C = 3
OUT_C = 8
KSIZE = 5
IMG = 32
POOL_HW = 14
FC_OUT = 10
F_PAD = 16
TB = 256

GW = 16
NW = 2
GROW = IN_C * GW
WK = 5 * GROW
WH = IMG * GROW
WM = 128
JL = 8


def _window_weights(Wc):
    m = np.arange(WM)
    xpar = m // 64
    jl = (m // OUT_C) % JL
    o = m % OUT_C
    k = np.arange(WK)
    yloc = k // GROW
    c = (k // GW) % IN_C
    dx = k % GW
    ws = []
    for w in range(NW):
        kx = (4 * w + dx)[None, :] - (2 * jl + xpar)[:, None]
        valid = (kx >= 0) & (kx < KSIZE)
        src = ((o[:, None] * IN_C + c[None, :]) * KSIZE + yloc[None, :]) * KSIZE \
            + np.clip(kx, 0, KSIZE - 1)
        wb = jnp.where(jnp.asarray(valid), Wc.reshape(-1)[jnp.asarray(src)], 0.0)
        ws.append(wb.astype(jnp.bfloat16))
    return jnp.stack(ws)


def _fc_weight(Wf):
    w4 = Wf.reshape(FC_OUT, OUT_C, POOL_HW, POOL_HW)
    w4 = jnp.transpose(w4, (2, 0, 3, 1)).reshape(POOL_HW, FC_OUT, 112)
    w4 = jnp.pad(w4, ((0, 0), (0, F_PAD - FC_OUT), (0, 0)))
    return w4.astype(jnp.bfloat16)


def _net_kernel(x_ref, w_ref, bct_ref, wf_ref, bf_ref, out_ref):
    w0 = w_ref[0]
    w1 = w_ref[1]
    bcA = bct_ref[0:48]
    bcB = bct_ref[48:64]
    accs = [jnp.zeros(out_ref.shape, jnp.float32) for _ in range(2)]
    for i in range(POOL_HW):
        r = []
        for w in range(NW):
            wm = w0 if w == 0 else w1
            for ypar in range(2):
                base = w * WH + 96 * i + 48 * ypar
                r.append(jnp.dot(wm, x_ref[0, base:base + WK, :],
                                 preferred_element_type=jnp.float32))
        r00, r01, r10, r11 = r
        mA = jnp.maximum(jnp.maximum(r00[0:48], r00[64:112]),
                         jnp.maximum(r01[0:48], r01[64:112]))
        mC = jnp.maximum(jnp.maximum(r10[16:64], r10[80:128]),
                         jnp.maximum(r11[16:64], r11[80:128]))
        s0 = jnp.maximum(r00[48:64] + r10[0:16], r00[112:128] + r10[64:80])
        s1 = jnp.maximum(r01[48:64] + r11[0:16], r01[112:128] + r11[64:80])
        mB = jnp.maximum(s0, s1)
        a = jnp.concatenate([
            jnp.maximum(mA + bcA, 0.0).astype(jnp.bfloat16),
            jnp.maximum(mB + bcB, 0.0).astype(jnp.bfloat16),
            jnp.maximum(mC + bcA, 0.0).astype(jnp.bfloat16),
        ], axis=0)
        accs[i % 2] = accs[i % 2] + jnp.dot(
            wf_ref[i], a, preferred_element_type=jnp.float32)
    out_ref[...] = accs[0] + accs[1] + bf_ref[...]


@jax.jit
def _forward(x, Wc, bc, Wf, bf):
    B = x.shape[0]
    grid = pl.cdiv(B, TB)
    Bp = grid * TB

    xb = x.astype(jnp.bfloat16)
    if Bp != B:
        xb = jnp.pad(xb, ((0, Bp - B), (0, 0), (0, 0), (0, 0)))
    xt = xb.reshape(grid, TB, IN_C, IMG, NW, GW)
    xt = jnp.transpose(xt, (0, 4, 3, 2, 5, 1)).reshape(grid, NW * WH, TB)

    wcw = _window_weights(Wc)
    bcf = bc.astype(jnp.float32)
    bct = jnp.concatenate([jnp.tile(bcf, 6), jnp.tile(bcf, 2)]).reshape(64, 1)
    wf_r = _fc_weight(Wf)
    bf_col = jnp.pad(bf.astype(jnp.float32), (0, F_PAD - FC_OUT)).reshape(F_PAD, 1)

    flops = 2 * Bp * POOL_HW * (4 * WM * WK + F_PAD * 112)
    bytes_accessed = (grid * NW * WH * TB * 2 + NW * WM * WK * 2
                      + POOL_HW * F_PAD * 112 * 2 + 64 * 4 + F_PAD * 4
                      + F_PAD * Bp * 4)

    out = pl.pallas_call(
        _net_kernel,
        out_shape=jax.ShapeDtypeStruct((F_PAD, Bp), jnp.float32),
        grid=(grid,),
        in_specs=[
            pl.BlockSpec((1, NW * WH, TB), lambda b: (b, 0, 0)),
            pl.BlockSpec((NW, WM, WK), lambda b: (0, 0, 0)),
            pl.BlockSpec((64, 1), lambda b: (0, 0)),
            pl.BlockSpec((POOL_HW, F_PAD, 112), lambda b: (0, 0, 0)),
            pl.BlockSpec((F_PAD, 1), lambda b: (0, 0)),
        ],
        out_specs=pl.BlockSpec((F_PAD, TB), lambda b: (0, b)),
        compiler_params=pltpu.CompilerParams(
            dimension_semantics=("parallel",),
        ),
        cost_estimate=pl.CostEstimate(flops=int(flops), transcendentals=0,
                                      bytes_accessed=int(bytes_accessed)),
    )(xt, wcw, bct, wf_r, bf_col)
    return jnp.transpose(out[:FC_OUT, :B])


def kernel(x, Wc, bc, Wf, bf):
    return _forward(x, Wc, bc, Wf, bf)
